# baseline (speedup 1.0000x reference)
import functools

import jax
import jax.numpy as jnp
from jax import lax
from jax.experimental import pallas as pl
from jax.experimental.pallas import tpu as pltpu
from jax.experimental.pallas import tpu_sc as plsc

EMBED_DIM = 64
LANES = 16
TILE_W = 128
CHUNK = 1
SLOTS = 7
PREF = 6


@functools.lru_cache(maxsize=None)
def _make_kernel(batch, num_rows):
    info = plsc.get_sparse_core_info()
    num_cores, num_subcores = info.num_cores, info.num_subcores
    num_workers = num_cores * num_subcores
    b_per_w = batch // num_workers
    n_chunks = b_per_w // CHUNK
    assert b_per_w % CHUNK == 0
    mesh = plsc.VectorSubcoreMesh(core_axis_name="c", subcore_axis_name="s")

    lane_iota = lambda: lax.iota(jnp.int32, LANES)

    @functools.partial(
        pl.kernel,
        mesh=mesh,
        out_type=jax.ShapeDtypeStruct((batch,), jnp.float32),
        compiler_params=pltpu.CompilerParams(
            use_tc_tiling_on_sc=True, needs_layout_passes=False),
        scratch_types=[
            pltpu.VMEM((b_per_w,), jnp.int32),
            pltpu.VMEM((b_per_w,), jnp.int32),
            pltpu.VMEM((b_per_w,), jnp.int32),
            pltpu.VMEM((b_per_w,), jnp.int32),
            pltpu.VMEM((SLOTS, CHUNK, EMBED_DIM, TILE_W), jnp.float32),
            pltpu.VMEM((SLOTS, CHUNK, EMBED_DIM, TILE_W), jnp.float32),
            pltpu.VMEM((b_per_w,), jnp.float32),
        ] + [pltpu.SemaphoreType.DMA] * SLOTS,
    )
    def mf_bpr(uid_hbm, iid_hbm, utT_hbm, itT_hbm, out_hbm,
               utoff_v, itoff_v, ulane_v, ilane_v, ubuf_v, ibuf_v,
               scores_v, *sems):
        wid = lax.axis_index("s") * num_cores + lax.axis_index("c")
        base = wid * b_per_w
        pltpu.sync_copy(uid_hbm.at[pl.ds(base, b_per_w)], utoff_v)
        pltpu.sync_copy(iid_hbm.at[pl.ds(base, b_per_w)], itoff_v)

        def split(v, _):
            s = pl.ds(v * LANES, LANES)
            ru = utoff_v[s]
            ri = itoff_v[s]
            ulane_v[s] = ru & (TILE_W - 1)
            ilane_v[s] = ri & (TILE_W - 1)
            utoff_v[s] = ru - (ru & (TILE_W - 1))
            itoff_v[s] = ri - (ri & (TILE_W - 1))
            return _

        lax.fori_loop(0, b_per_w // LANES, split, 0)


        def splat_at(ref, i):
            return plsc.load_gather(ref, [jnp.full((LANES,), i, jnp.int32)])

        def fire(c, slot):
            sem = sems[slot]
            for j in range(CHUNK):
                i = c * CHUNK + j
                tu = splat_at(utoff_v, i)[0]
                ti = splat_at(itoff_v, i)[0]
                pltpu.async_copy(
                    utT_hbm.at[:, pl.ds(pl.multiple_of(tu, TILE_W), TILE_W)],
                    ubuf_v.at[slot, j], sem)
                pltpu.async_copy(
                    itT_hbm.at[:, pl.ds(pl.multiple_of(ti, TILE_W), TILE_W)],
                    ibuf_v.at[slot, j], sem)

        def drain(slot):
            sem = sems[slot]
            for j in range(CHUNK):
                pltpu.make_async_copy(
                    utT_hbm.at[:, pl.ds(0, TILE_W)], ubuf_v.at[slot, j], sem
                ).wait()
                pltpu.make_async_copy(
                    itT_hbm.at[:, pl.ds(0, TILE_W)], ibuf_v.at[slot, j], sem
                ).wait()

        def extract(c, slot):
            lane0 = lane_iota() == 0
            for j in range(CHUNK):
                i = c * CHUNK + j
                ul = splat_at(ulane_v, i)
                il = splat_at(ilane_v, i)
                acc = None
                for cb in range(EMBED_DIM // LANES):
                    cvec = cb * LANES + lane_iota()
                    u = plsc.load_gather(ubuf_v.at[slot, j], [cvec, ul])
                    v = plsc.load_gather(ibuf_v.at[slot, j], [cvec, il])
                    acc = u * v if acc is None else acc + u * v
                score = jnp.full((LANES,), jnp.sum(acc), jnp.float32)
                plsc.store_scatter(
                    scores_v, [jnp.full((LANES,), i, jnp.int32)], score,
                    mask=lane0)

        for k in range(PREF):
            fire(k, k)

        def step(c, fire_slot, dx_slot):
            @pl.when(c + PREF < n_chunks)
            def _fire_next():
                fire(c + PREF, fire_slot)

            @pl.when(c < n_chunks)
            def _dx():
                drain(dx_slot)
                extract(c, dx_slot)

        def body(p, _):
            c = p * SLOTS
            for q in range(SLOTS):
                step(c + q, (q + PREF) % SLOTS, q)
            return _

        lax.fori_loop(0, (n_chunks + SLOTS - 1) // SLOTS, body, 0)
        pltpu.sync_copy(scores_v, out_hbm.at[pl.ds(base, b_per_w)])

    return mf_bpr


def kernel(user_ids, item_ids, user_table, item_table):
    batch = user_ids.shape[0]
    k = _make_kernel(batch, user_table.shape[0])
    return k(user_ids, item_ids, user_table.T, item_table.T)


# speedup vs baseline: 1.0033x; 1.0033x over previous
import functools

import jax
import jax.numpy as jnp
from jax import lax
from jax.experimental import pallas as pl
from jax.experimental.pallas import tpu as pltpu
from jax.experimental.pallas import tpu_sc as plsc

EMBED_DIM = 64
LANES = 16
TILE_W = 128
CHUNK = 1
SLOTS = 7
PREF = 5


@functools.lru_cache(maxsize=None)
def _make_kernel(batch, num_rows):
    info = plsc.get_sparse_core_info()
    num_cores, num_subcores = info.num_cores, info.num_subcores
    num_workers = num_cores * num_subcores
    b_per_w = batch // num_workers
    n_chunks = b_per_w // CHUNK
    assert b_per_w % CHUNK == 0
    mesh = plsc.VectorSubcoreMesh(core_axis_name="c", subcore_axis_name="s")

    lane_iota = lambda: lax.iota(jnp.int32, LANES)

    @functools.partial(
        pl.kernel,
        mesh=mesh,
        out_type=jax.ShapeDtypeStruct((batch,), jnp.float32),
        compiler_params=pltpu.CompilerParams(
            use_tc_tiling_on_sc=True, needs_layout_passes=False),
        scratch_types=[
            pltpu.VMEM((b_per_w,), jnp.int32),
            pltpu.VMEM((b_per_w,), jnp.int32),
            pltpu.VMEM((b_per_w,), jnp.int32),
            pltpu.VMEM((b_per_w,), jnp.int32),
            pltpu.VMEM((SLOTS, CHUNK, EMBED_DIM, TILE_W), jnp.float32),
            pltpu.VMEM((SLOTS, CHUNK, EMBED_DIM, TILE_W), jnp.float32),
            pltpu.VMEM((b_per_w,), jnp.float32),
        ] + [pltpu.SemaphoreType.DMA] * SLOTS,
    )
    def mf_bpr(uid_hbm, iid_hbm, utT_hbm, itT_hbm, out_hbm,
               utoff_v, itoff_v, ulane_v, ilane_v, ubuf_v, ibuf_v,
               scores_v, *sems):
        wid = lax.axis_index("s") * num_cores + lax.axis_index("c")
        base = wid * b_per_w
        pltpu.sync_copy(uid_hbm.at[pl.ds(base, b_per_w)], utoff_v)
        pltpu.sync_copy(iid_hbm.at[pl.ds(base, b_per_w)], itoff_v)

        def split(v, _):
            s = pl.ds(v * LANES, LANES)
            ru = utoff_v[s]
            ri = itoff_v[s]
            ulane_v[s] = ru & (TILE_W - 1)
            ilane_v[s] = ri & (TILE_W - 1)
            utoff_v[s] = ru - (ru & (TILE_W - 1))
            itoff_v[s] = ri - (ri & (TILE_W - 1))
            return _

        lax.fori_loop(0, b_per_w // LANES, split, 0)


        def splat_at(ref, i):
            return plsc.load_gather(ref, [jnp.full((LANES,), i, jnp.int32)])

        def fire(c, slot):
            sem = sems[slot]
            for j in range(CHUNK):
                i = c * CHUNK + j
                tu = splat_at(utoff_v, i)[0]
                ti = splat_at(itoff_v, i)[0]
                pltpu.async_copy(
                    utT_hbm.at[:, pl.ds(pl.multiple_of(tu, TILE_W), TILE_W)],
                    ubuf_v.at[slot, j], sem)
                pltpu.async_copy(
                    itT_hbm.at[:, pl.ds(pl.multiple_of(ti, TILE_W), TILE_W)],
                    ibuf_v.at[slot, j], sem)

        def drain(slot):
            sem = sems[slot]
            for j in range(CHUNK):
                pltpu.make_async_copy(
                    utT_hbm.at[:, pl.ds(0, TILE_W)], ubuf_v.at[slot, j], sem
                ).wait()
                pltpu.make_async_copy(
                    itT_hbm.at[:, pl.ds(0, TILE_W)], ibuf_v.at[slot, j], sem
                ).wait()

        def extract(c, slot):
            lane0 = lane_iota() == 0
            for j in range(CHUNK):
                i = c * CHUNK + j
                ul = splat_at(ulane_v, i)
                il = splat_at(ilane_v, i)
                acc = None
                for cb in range(EMBED_DIM // LANES):
                    cvec = cb * LANES + lane_iota()
                    u = plsc.load_gather(ubuf_v.at[slot, j], [cvec, ul])
                    v = plsc.load_gather(ibuf_v.at[slot, j], [cvec, il])
                    acc = u * v if acc is None else acc + u * v
                score = jnp.full((LANES,), jnp.sum(acc), jnp.float32)
                plsc.store_scatter(
                    scores_v, [jnp.full((LANES,), i, jnp.int32)], score,
                    mask=lane0)

        for k in range(PREF):
            fire(k, k)

        def step(c, fire_slot, dx_slot):
            @pl.when(c + PREF < n_chunks)
            def _fire_next():
                fire(c + PREF, fire_slot)

            @pl.when(c < n_chunks)
            def _dx():
                drain(dx_slot)
                extract(c, dx_slot)

        def body(p, _):
            c = p * SLOTS
            for q in range(SLOTS):
                step(c + q, (q + PREF) % SLOTS, q)
            return _

        lax.fori_loop(0, (n_chunks + SLOTS - 1) // SLOTS, body, 0)
        pltpu.sync_copy(scores_v, out_hbm.at[pl.ds(base, b_per_w)])

    return mf_bpr


def kernel(user_ids, item_ids, user_table, item_table):
    batch = user_ids.shape[0]
    k = _make_kernel(batch, user_table.shape[0])
    return k(user_ids, item_ids, user_table.T, item_table.T)
